# baseline (speedup 1.0000x reference)
import functools

import jax
import jax.numpy as jnp
from jax import lax
from jax.experimental import pallas as pl
from jax.experimental.pallas import tpu as pltpu
from jax.experimental.pallas import tpu_sc as plsc

N = 10000
E = 320000
F = 128
H = 128
C = 40
CP = 48

NC = 2
NS = 16
NW = NC * NS
EW = E // NW
RPT = 624
TAIL0 = NS * RPT
TAIL = N - TAIL0


@functools.lru_cache(maxsize=None)
def _make_agg(D, with_deg, K):
    NCHUNK = EW // K
    mesh = plsc.VectorSubcoreMesh(
        core_axis_name="c", subcore_axis_name="s",
        num_cores=NC, num_subcores=NS)
    if with_deg:
        out_type = (jax.ShapeDtypeStruct((NC, N, D), jnp.float32),
                    jax.ShapeDtypeStruct((NC, N, 16), jnp.float32))
    else:
        out_type = jax.ShapeDtypeStruct((NC, N, D), jnp.float32)
    scratch = [
        pltpu.VMEM((NCHUNK, K), jnp.int32),
        pltpu.VMEM((NCHUNK, K), jnp.int32),
        pltpu.VMEM((K, D), jnp.float32),
        pltpu.VMEM((K, D), jnp.float32),
        pltpu.VMEM_SHARED((N, D), jnp.float32),
        pltpu.SemaphoreType.DMA,
        pltpu.SemaphoreType.DMA,
        pltpu.SemaphoreType.DMA,
        pltpu.SemaphoreType.DMA,
        pltpu.SemaphoreType.DMA,
    ]
    if with_deg:
        scratch += [
            pltpu.VMEM((K, 16), jnp.float32),
            pltpu.VMEM_SHARED((N, 16), jnp.float32),
        ]

    def body(*refs):
        if with_deg:
            (y, srcs, dsts, zrows, zdeg, ones_h,
             out, deg_out, src_v, dst_v, rows0, rows1, acc,
             sem0, sem1, ssem0, ssem1, dsem,
             ones_v, dacc) = refs
        else:
            (y, srcs, dsts, zrows,
             out, src_v, dst_v, rows0, rows1, acc,
             sem0, sem1, ssem0, ssem1, dsem) = refs
        c = lax.axis_index("c")
        s = lax.axis_index("s")
        wid = s * NC + c
        r0 = s * RPT
        pltpu.sync_copy(zrows.at[pl.ds(r0, RPT)], acc.at[pl.ds(r0, RPT)])
        if with_deg:
            pltpu.sync_copy(zdeg.at[pl.ds(r0, RPT)], dacc.at[pl.ds(r0, RPT)])
            pltpu.sync_copy(ones_h, ones_v)

        @pl.when(s == NS - 1)
        def _init_tail():
            pltpu.sync_copy(zrows.at[pl.ds(TAIL0, TAIL)],
                            acc.at[pl.ds(TAIL0, TAIL)])
            if with_deg:
                pltpu.sync_copy(zdeg.at[pl.ds(TAIL0, TAIL)],
                                dacc.at[pl.ds(TAIL0, TAIL)])

        pltpu.sync_copy(srcs.at[wid], src_v)
        pltpu.sync_copy(dsts.at[wid], dst_v)
        plsc.subcore_barrier()

        def gather(i, buf, sem):
            return pltpu.async_copy(y.at[src_v.at[i]], buf, sem)

        def wait_gather(i, buf, sem):
            pltpu.make_async_copy(y.at[src_v.at[i]], buf, sem).wait()

        def scatter_sync(i, buf):
            pltpu.sync_copy(buf, acc.at[dst_v.at[i]], add=True)
            if with_deg:
                pltpu.sync_copy(ones_v, dacc.at[dst_v.at[i]], add=True)

        gather(0, rows0, sem0)
        gather(1, rows1, sem1)

        def pair(j, carry):
            e = 2 * j
            wait_gather(e, rows0, sem0)
            pltpu.async_copy(rows0, acc.at[dst_v.at[e]], ssem0, add=True)
            if with_deg:
                pltpu.async_copy(ones_v, dacc.at[dst_v.at[e]], dsem, add=True)
            wait_gather(e + 1, rows1, sem1)
            pltpu.async_copy(rows1, acc.at[dst_v.at[e + 1]], ssem1, add=True)
            if with_deg:
                pltpu.async_copy(ones_v, dacc.at[dst_v.at[e + 1]], dsem,
                                 add=True)

            pltpu.make_async_copy(rows0, acc.at[dst_v.at[e]], ssem0).wait()

            @pl.when(e + 2 < NCHUNK)
            def _():
                gather(e + 2, rows0, sem0)

            pltpu.make_async_copy(rows1, acc.at[dst_v.at[e + 1]], ssem1).wait()

            @pl.when(e + 3 < NCHUNK)
            def _():
                gather(e + 3, rows1, sem1)

            if with_deg:
                pltpu.make_async_copy(ones_v, dacc.at[dst_v.at[e]],
                                      dsem).wait()
                pltpu.make_async_copy(ones_v, dacc.at[dst_v.at[e + 1]],
                                      dsem).wait()
            return carry

        lax.fori_loop(0, NCHUNK // 2, pair, 0)
        if NCHUNK % 2:
            last = NCHUNK - 1
            wait_gather(last, rows0, sem0)
            scatter_sync(last, rows0)
        plsc.subcore_barrier()
        pltpu.sync_copy(acc.at[pl.ds(r0, RPT)], out.at[c, pl.ds(r0, RPT)])
        if with_deg:
            pltpu.sync_copy(dacc.at[pl.ds(r0, RPT)],
                            deg_out.at[c, pl.ds(r0, RPT)])

        @pl.when(s == NS - 1)
        def _flush_tail():
            pltpu.sync_copy(acc.at[pl.ds(TAIL0, TAIL)],
                            out.at[c, pl.ds(TAIL0, TAIL)])
            if with_deg:
                pltpu.sync_copy(dacc.at[pl.ds(TAIL0, TAIL)],
                                deg_out.at[c, pl.ds(TAIL0, TAIL)])

    return pl.kernel(body, out_type=out_type, mesh=mesh,
                     scratch_types=scratch,
                     compiler_params=pltpu.CompilerParams(
                         use_tc_tiling_on_sc=False))


_BM = 1000


def _tc1_body(x, wl, wr, bl, y1, r1b):
    xv = x[...]
    y1[...] = jnp.dot(xv, wl[...], preferred_element_type=jnp.float32)
    r1b[...] = jnp.dot(xv, wr[...], preferred_element_type=jnp.float32) + bl[...]


def _tc2_body(a1, degp, r1b, wl2, wr2, bl2, y2, r2b, deg):
    agg = a1[0] + a1[1]
    d = jnp.maximum(degp[0] + degp[1], 1.0)
    pre = agg / d[:, 0:1] + r1b[...]
    h = jnp.where(pre > 0, pre, jnp.exp(jnp.minimum(pre, 0.0)) - 1.0)
    y2[...] = jnp.dot(h, wl2[...], preferred_element_type=jnp.float32)
    r2b[...] = jnp.dot(h, wr2[...], preferred_element_type=jnp.float32) + bl2[...]
    deg[...] = d


def _tc3_body(a2, deg, r2b, out):
    out[...] = (a2[0, :, :C] + a2[1, :, :C]) / deg[:, 0:1] + r2b[...]


def kernel(x, edge_index, Wl1, bl1, Wr1, Wl2, bl2, Wr2, Q, epoch):
    K1, K2 = 40, 80
    src1 = edge_index[0].reshape(NW, EW // K1, K1)
    dst1 = edge_index[1].reshape(NW, EW // K1, K1)
    src2 = edge_index[0].reshape(NW, EW // K2, K2)
    dst2 = edge_index[1].reshape(NW, EW // K2, K2)
    f32 = jnp.float32

    nblk = N // _BM
    row_spec = lambda w: pl.BlockSpec((_BM, w), lambda i: (i, 0))
    full = lambda shape: pl.BlockSpec(shape, lambda i: tuple(0 for _ in shape))
    part_spec = lambda w: pl.BlockSpec((NC, _BM, w), lambda i: (0, i, 0))

    y1, r1b = pl.pallas_call(
        _tc1_body,
        grid=(nblk,),
        in_specs=[row_spec(F), full((F, H)), full((F, H)), full((1, H))],
        out_specs=[row_spec(H), row_spec(H)],
        out_shape=[jax.ShapeDtypeStruct((N, H), f32),
                   jax.ShapeDtypeStruct((N, H), f32)],
    )(x, Wl1, Wr1, bl1.reshape(1, H))

    zrows = jnp.zeros((N, H), f32)
    zdeg = jnp.zeros((N, 16), f32)
    ones_h = jnp.ones((40, 16), f32)
    a1, degp = _make_agg(H, True, K1)(y1, src1, dst1, zrows, zdeg, ones_h)

    Wl2p = jnp.pad(Wl2, ((0, 0), (0, CP - C)))
    y2, r2b, deg = pl.pallas_call(
        _tc2_body,
        grid=(nblk,),
        in_specs=[part_spec(H), part_spec(16), row_spec(H),
                  full((H, CP)), full((H, C)), full((1, C))],
        out_specs=[row_spec(CP), row_spec(C), row_spec(16)],
        out_shape=[jax.ShapeDtypeStruct((N, CP), f32),
                   jax.ShapeDtypeStruct((N, C), f32),
                   jax.ShapeDtypeStruct((N, 16), f32)],
    )(a1, degp, r1b, Wl2p, Wr2, bl2.reshape(1, C))

    zrows2 = jnp.zeros((N, CP), f32)
    a2 = _make_agg(CP, False, K2)(y2, src2, dst2, zrows2)

    out = pl.pallas_call(
        _tc3_body,
        grid=(nblk,),
        in_specs=[part_spec(CP), row_spec(16), row_spec(C)],
        out_specs=row_spec(C),
        out_shape=jax.ShapeDtypeStruct((N, C), f32),
    )(a2, deg, r2b)

    return (out, Q)


# speedup vs baseline: 1.2995x; 1.2995x over previous
import functools

import jax
import jax.numpy as jnp
from jax import lax
from jax.experimental import pallas as pl
from jax.experimental.pallas import tpu as pltpu
from jax.experimental.pallas import tpu_sc as plsc

N = 10000
E = 320000
F = 128
H = 128
C = 40
CP = 48

NC = 2
NS = 16
NW = NC * NS
EW = E // NW
RPT = 624
TAIL0 = NS * RPT
TAIL = N - TAIL0


@functools.lru_cache(maxsize=None)
def _make_agg(D, with_deg, K, NBUF):
    NCHUNK = EW // K
    mesh = plsc.VectorSubcoreMesh(
        core_axis_name="c", subcore_axis_name="s",
        num_cores=NC, num_subcores=NS)
    if with_deg:
        out_type = (jax.ShapeDtypeStruct((NC, N, D), jnp.float32),
                    jax.ShapeDtypeStruct((NC, N, 16), jnp.float32))
    else:
        out_type = jax.ShapeDtypeStruct((NC, N, D), jnp.float32)
    scratch = [
        pltpu.VMEM((NCHUNK, K), jnp.int32),
        pltpu.VMEM((NCHUNK, K), jnp.int32),
    ] + [pltpu.VMEM((K, D), jnp.float32) for _ in range(NBUF)] + [
        pltpu.VMEM_SHARED((N, D), jnp.float32),
    ] + [pltpu.SemaphoreType.DMA for _ in range(NBUF + 1)]
    if with_deg:
        scratch += [
            pltpu.VMEM((K, 16), jnp.float32),
            pltpu.VMEM_SHARED((N, 16), jnp.float32),
        ]

    def body(*refs):
        if with_deg:
            (y, srcs, dsts, zrows, zdeg, ones_h, out, deg_out,
             src_v, dst_v, *rest) = refs
            bufs, (acc,), sems, dsem, (ones_v, dacc) = (
                rest[:NBUF], rest[NBUF:NBUF + 1],
                rest[NBUF + 1:2 * NBUF + 1], rest[2 * NBUF + 1],
                rest[2 * NBUF + 2:])
        else:
            (y, srcs, dsts, zrows, out, src_v, dst_v, *rest) = refs
            bufs, (acc,), sems, dsem = (
                rest[:NBUF], rest[NBUF:NBUF + 1],
                rest[NBUF + 1:2 * NBUF + 1], rest[2 * NBUF + 1])
        c = lax.axis_index("c")
        s = lax.axis_index("s")
        wid = s * NC + c
        r0 = s * RPT
        pltpu.sync_copy(zrows.at[pl.ds(r0, RPT)], acc.at[pl.ds(r0, RPT)])
        if with_deg:
            pltpu.sync_copy(zdeg.at[pl.ds(r0, RPT)], dacc.at[pl.ds(r0, RPT)])
            pltpu.sync_copy(ones_h, ones_v)

        @pl.when(s == NS - 1)
        def _init_tail():
            pltpu.sync_copy(zrows.at[pl.ds(TAIL0, TAIL)],
                            acc.at[pl.ds(TAIL0, TAIL)])
            if with_deg:
                pltpu.sync_copy(zdeg.at[pl.ds(TAIL0, TAIL)],
                                dacc.at[pl.ds(TAIL0, TAIL)])

        pltpu.sync_copy(srcs.at[wid], src_v)
        pltpu.sync_copy(dsts.at[wid], dst_v)
        plsc.subcore_barrier()

        def gather(i, b):
            pltpu.async_copy(y.at[src_v.at[i]], bufs[b], sems[b])

        def slot(i, b):
            pltpu.make_async_copy(y.at[src_v.at[i]], bufs[b], sems[b]).wait()
            pltpu.sync_copy(bufs[b], acc.at[dst_v.at[i]], add=True)
            if with_deg:
                pltpu.async_copy(ones_v, dacc.at[dst_v.at[i]], dsem, add=True)

            @pl.when(i + NBUF < NCHUNK)
            def _():
                gather(i + NBUF, b)

            if with_deg:
                pltpu.make_async_copy(ones_v, dacc.at[dst_v.at[i]],
                                      dsem).wait()

        for b in range(NBUF):
            gather(b, b)

        def turn(t, carry):
            for b in range(NBUF):
                slot(t * NBUF + b, b)
            return carry

        lax.fori_loop(0, NCHUNK // NBUF, turn, 0)
        for r in range(NCHUNK % NBUF):
            slot(NBUF * (NCHUNK // NBUF) + r, r)
        plsc.subcore_barrier()
        pltpu.sync_copy(acc.at[pl.ds(r0, RPT)], out.at[c, pl.ds(r0, RPT)])
        if with_deg:
            pltpu.sync_copy(dacc.at[pl.ds(r0, RPT)],
                            deg_out.at[c, pl.ds(r0, RPT)])

        @pl.when(s == NS - 1)
        def _flush_tail():
            pltpu.sync_copy(acc.at[pl.ds(TAIL0, TAIL)],
                            out.at[c, pl.ds(TAIL0, TAIL)])
            if with_deg:
                pltpu.sync_copy(dacc.at[pl.ds(TAIL0, TAIL)],
                                deg_out.at[c, pl.ds(TAIL0, TAIL)])

    return pl.kernel(body, out_type=out_type, mesh=mesh,
                     scratch_types=scratch,
                     compiler_params=pltpu.CompilerParams(
                         use_tc_tiling_on_sc=False))


_BM = 1000


def _tc1_body(x, wl, wr, bl, y1, r1b):
    xv = x[...]
    y1[...] = jnp.dot(xv, wl[...], preferred_element_type=jnp.float32)
    r1b[...] = jnp.dot(xv, wr[...], preferred_element_type=jnp.float32) + bl[...]


def _tc2_body(a1, degp, r1b, wl2, wr2, bl2, y2, r2b, deg):
    agg = a1[0] + a1[1]
    d = jnp.maximum(degp[0] + degp[1], 1.0)
    pre = agg / d[:, 0:1] + r1b[...]
    h = jnp.where(pre > 0, pre, jnp.exp(jnp.minimum(pre, 0.0)) - 1.0)
    y2[...] = jnp.dot(h, wl2[...], preferred_element_type=jnp.float32)
    r2b[...] = jnp.dot(h, wr2[...], preferred_element_type=jnp.float32) + bl2[...]
    deg[...] = d


def _tc3_body(a2, deg, r2b, out):
    out[...] = (a2[0, :, :C] + a2[1, :, :C]) / deg[:, 0:1] + r2b[...]


def kernel(x, edge_index, Wl1, bl1, Wr1, Wl2, bl2, Wr2, Q, epoch):
    K1, K2 = 40, 80
    src1 = edge_index[0].reshape(NW, EW // K1, K1)
    dst1 = edge_index[1].reshape(NW, EW // K1, K1)
    src2 = edge_index[0].reshape(NW, EW // K2, K2)
    dst2 = edge_index[1].reshape(NW, EW // K2, K2)
    f32 = jnp.float32

    nblk = N // _BM
    row_spec = lambda w: pl.BlockSpec((_BM, w), lambda i: (i, 0))
    full = lambda shape: pl.BlockSpec(shape, lambda i: tuple(0 for _ in shape))
    part_spec = lambda w: pl.BlockSpec((NC, _BM, w), lambda i: (0, i, 0))

    y1, r1b = pl.pallas_call(
        _tc1_body,
        grid=(nblk,),
        in_specs=[row_spec(F), full((F, H)), full((F, H)), full((1, H))],
        out_specs=[row_spec(H), row_spec(H)],
        out_shape=[jax.ShapeDtypeStruct((N, H), f32),
                   jax.ShapeDtypeStruct((N, H), f32)],
    )(x, Wl1, Wr1, bl1.reshape(1, H))

    zrows = jnp.zeros((N, H), f32)
    zdeg = jnp.zeros((N, 16), f32)
    ones_h = jnp.ones((40, 16), f32)
    a1, degp = _make_agg(H, True, K1, 3)(y1, src1, dst1, zrows, zdeg, ones_h)

    Wl2p = jnp.pad(Wl2, ((0, 0), (0, CP - C)))
    y2, r2b, deg = pl.pallas_call(
        _tc2_body,
        grid=(nblk,),
        in_specs=[part_spec(H), part_spec(16), row_spec(H),
                  full((H, CP)), full((H, C)), full((1, C))],
        out_specs=[row_spec(CP), row_spec(C), row_spec(16)],
        out_shape=[jax.ShapeDtypeStruct((N, CP), f32),
                   jax.ShapeDtypeStruct((N, C), f32),
                   jax.ShapeDtypeStruct((N, 16), f32)],
    )(a1, degp, r1b, Wl2p, Wr2, bl2.reshape(1, C))

    zrows2 = jnp.zeros((N, CP), f32)
    a2 = _make_agg(CP, False, K2, 4)(y2, src2, dst2, zrows2)

    out = pl.pallas_call(
        _tc3_body,
        grid=(nblk,),
        in_specs=[part_spec(CP), row_spec(16), row_spec(C)],
        out_specs=row_spec(C),
        out_shape=jax.ShapeDtypeStruct((N, C), f32),
    )(a2, deg, r2b)

    return (out, Q)
